# baseline (speedup 1.0000x reference)
import functools
import math

import jax
import jax.numpy as jnp
from jax.experimental import pallas as pl
from jax.experimental.pallas import tpu as pltpu

_EPS = 1e-6
_LANES = 128


def _loss_body(act_ref, cand_ref, inp_ref, enc_ref, ins_ref, mask_ref,
               out_ref, acc_ref, *, scale, blk):
    i = pl.program_id(0)
    j = pl.program_id(1)
    nb = pl.num_programs(1)

    @pl.when(j == 0)
    def _init():
        acc_ref[0] = 0.0
        acc_ref[1] = 0.0
        acc_ref[2] = 0.0
        acc_ref[3] = 0.0
        acc_ref[4] = 0.0

    lane = jax.lax.rem(act_ref[i], _LANES)
    onehot = (jax.lax.broadcasted_iota(jnp.int32, (1, _LANES), 1)
              == lane).astype(jnp.float32)
    sel_row = jax.lax.dot_general(
        onehot, inp_ref[0],
        dimension_numbers=(((1,), (1,)), ((), ())),
        preferred_element_type=jnp.float32)

    logits = jax.lax.dot_general(
        sel_row, enc_ref[0],
        dimension_numbers=(((1,), (1,)), ((), ())),
        preferred_element_type=jnp.float32)

    maskb = (mask_ref[0] > 0.5).astype(jnp.float32)
    gold = (ins_ref[0] > 0.5).astype(jnp.float32)
    pred = jax.nn.sigmoid(logits * scale)
    p = pred * maskb
    t = gold * maskb
    pt = p * t + (1.0 - p) * (1.0 - t)
    focal = -((1.0 - pt) ** 2) * jnp.log(pt + _EPS)

    acc_ref[0] += jnp.sum(focal * maskb)
    acc_ref[1] += jnp.sum(p * t)
    acc_ref[2] += jnp.sum(p)
    acc_ref[3] += jnp.sum(t)
    acc_ref[4] += jnp.sum(maskb)

    @pl.when(j == nb - 1)
    def _finish():
        mask_sum_e = acc_ref[4] + _EPS
        focal_loss = acc_ref[0] / mask_sum_e
        dice_loss = 1.0 - (2.0 * acc_ref[1] + _EPS) / (acc_ref[2] + acc_ref[3] + _EPS)
        loss_atten = (0.5 * focal_loss + dice_loss) * acc_ref[3]
        out_ref[0, 0, :] = jnp.full((_LANES,), loss_atten / mask_sum_e,
                                    dtype=jnp.float32)


@jax.jit
def kernel(input, encode, ins_seg, mask, actions, candidate_idx):
    b, c, h, w = input.shape
    hw = h * w
    n_ins = ins_seg.shape[1]
    blk = 2048
    nb = hw // blk

    inp_flat = input.reshape(b, c, hw)
    ins_flat = ins_seg.reshape(b * n_ins, 1, hw)
    mask3 = mask.reshape(b, 1, hw)

    grid_spec = pltpu.PrefetchScalarGridSpec(
        num_scalar_prefetch=2,
        grid=(b, nb),
        in_specs=[
            pl.BlockSpec((1, c, _LANES),
                         lambda i, j, act, cand: (i, 0, act[i] // _LANES)),
            pl.BlockSpec((1, blk, c), lambda i, j, act, cand: (i, j, 0)),
            pl.BlockSpec((1, 1, blk),
                         lambda i, j, act, cand: (i * n_ins + cand[i], 0, j)),
            pl.BlockSpec((1, 1, blk), lambda i, j, act, cand: (i, 0, j)),
        ],
        out_specs=pl.BlockSpec((1, 1, _LANES), lambda i, j, act, cand: (i, 0, 0)),
        scratch_shapes=[pltpu.SMEM((8,), jnp.float32)],
    )

    out = pl.pallas_call(
        functools.partial(_loss_body, scale=1.0 / math.sqrt(c), blk=blk),
        grid_spec=grid_spec,
        out_shape=jax.ShapeDtypeStruct((b, 1, _LANES), jnp.float32),
        compiler_params=pltpu.CompilerParams(
            dimension_semantics=("parallel", "arbitrary")),
    )(actions, candidate_idx, inp_flat, encode, ins_flat, mask3)
    return out[:, 0, 0]


# speedup vs baseline: 2.5964x; 2.5964x over previous
import functools
import math

import jax
import jax.numpy as jnp
from jax.experimental import pallas as pl
from jax.experimental.pallas import tpu as pltpu

_EPS = 1e-6


def _loss_body(act_ref, cand_ref, inp_hbm, enc_ref, ins_ref, mask_ref,
               out_ref, acc_ref, selbuf_ref, sem, *, blk, w, scale):
    i = pl.program_id(0)
    j = pl.program_id(1)
    nb = pl.num_programs(1)
    b = pl.num_programs(0)
    h_blk = blk // w

    @pl.when((i == 0) & (j == 0))
    def _fetch_selected():
        copies = [
            pltpu.make_async_copy(
                inp_hbm.at[ib, :, act_ref[ib] // w, :], selbuf_ref.at[ib], sem)
            for ib in range(b)
        ]
        for cp in copies:
            cp.start()
        for cp in copies:
            cp.wait()

    @pl.when(j == 0)
    def _init():
        acc_ref[0] = 0.0
        acc_ref[1] = 0.0
        acc_ref[2] = 0.0
        acc_ref[3] = 0.0
        acc_ref[4] = 0.0

    aw = jax.lax.rem(act_ref[i], w)
    onehot = (jax.lax.broadcasted_iota(jnp.int32, (1, w), 1)
              == aw).astype(jnp.float32)
    sel_row = jax.lax.dot_general(
        onehot, selbuf_ref[i],
        dimension_numbers=(((1,), (1,)), ((), ())),
        preferred_element_type=jnp.float32)

    logits = jax.lax.dot_general(
        sel_row, enc_ref[0],
        dimension_numbers=(((1,), (1,)), ((), ())),
        preferred_element_type=jnp.float32)
    logits = logits.reshape(h_blk, w)

    maskb = (mask_ref[0] > 0.5).astype(jnp.float32)
    gold = (ins_ref[0] > 0.5).astype(jnp.float32)
    pred = jax.nn.sigmoid(logits * scale)
    p = pred * maskb
    t = gold * maskb
    pt = p * t + (1.0 - p) * (1.0 - t)
    focal = -((1.0 - pt) ** 2) * jnp.log(pt + _EPS)

    acc_ref[0] += jnp.sum(focal * maskb)
    acc_ref[1] += jnp.sum(p * t)
    acc_ref[2] += jnp.sum(p)
    acc_ref[3] += jnp.sum(t)
    acc_ref[4] += jnp.sum(maskb)

    @pl.when(j == nb - 1)
    def _finish():
        mask_sum_e = acc_ref[4] + _EPS
        focal_loss = acc_ref[0] / mask_sum_e
        dice_loss = 1.0 - (2.0 * acc_ref[1] + _EPS) / (acc_ref[2] + acc_ref[3] + _EPS)
        loss_atten = (0.5 * focal_loss + dice_loss) * acc_ref[3]
        out_ref[0, 0, :] = jnp.full((w,), loss_atten / mask_sum_e,
                                    dtype=jnp.float32)


@jax.jit
def kernel(input, encode, ins_seg, mask, actions, candidate_idx):
    b, c, h, w = input.shape
    hw = h * w
    n_ins = ins_seg.shape[1]
    blk = 2048
    nb = hw // blk
    h_blk = blk // w

    ins2 = ins_seg.reshape(b, n_ins * h, w)
    mask3 = mask.reshape(b, h, w)

    grid_spec = pltpu.PrefetchScalarGridSpec(
        num_scalar_prefetch=2,
        grid=(b, nb),
        in_specs=[
            pl.BlockSpec(memory_space=pl.ANY),
            pl.BlockSpec((1, blk, c), lambda i, j, act, cand: (i, j, 0)),
            pl.BlockSpec((1, h_blk, w),
                         lambda i, j, act, cand:
                         (i, cand[i] * (h // h_blk) + j, 0)),
            pl.BlockSpec((1, h_blk, w), lambda i, j, act, cand: (i, j, 0)),
        ],
        out_specs=pl.BlockSpec((1, 1, w), lambda i, j, act, cand: (i, 0, 0)),
        scratch_shapes=[
            pltpu.SMEM((8,), jnp.float32),
            pltpu.VMEM((b, c, w), jnp.float32),
            pltpu.SemaphoreType.DMA,
        ],
    )

    out = pl.pallas_call(
        functools.partial(_loss_body, blk=blk, w=w, scale=1.0 / math.sqrt(c)),
        grid_spec=grid_spec,
        out_shape=jax.ShapeDtypeStruct((b, 1, w), jnp.float32),
    )(actions, candidate_idx, input, encode, ins2, mask3)
    return out[:, 0, 0]
